# baseline (speedup 1.0000x reference)
import math

import jax
import jax.numpy as jnp
from jax.experimental import pallas as pl
from jax.experimental.pallas import tpu as pltpu

_B, _H, _S, _D = 1, 16, 2048, 128
_BM = 192
_TOPK = 512
_RAND_P = 0.01
_LOCAL_W = 128
_G = -(-_S // _BM)
_SP = _G * _BM
_CH = 256


def _fused_kernel(q_ref, k_ref, v_ref, rm_ref, o_ref, oc_ref):
    g = pl.program_id(1)
    q = q_ref[0, 0]
    k = k_ref[0, 0]
    v = v_ref[0, 0]
    scale = 1.0 / math.sqrt(_D)

    logits = jax.lax.dot_general(
        q.astype(jnp.bfloat16), k.astype(jnp.bfloat16),
        (((1,), (1,)), ((), ())),
        preferred_element_type=jnp.float32) * scale
    m = jnp.max(logits, axis=-1, keepdims=True)
    p = jnp.exp(logits - m)
    l = jnp.sum(p, axis=-1, keepdims=True)
    vb = v.astype(jnp.bfloat16)
    o = jax.lax.dot_general(
        p.astype(jnp.bfloat16), vb, (((1,), (0,)), ((), ())),
        preferred_element_type=jnp.float32) / l
    o_ref[0, 0] = o

    rowT = jax.lax.broadcasted_iota(jnp.int32, (1, _BM), 1)
    validT = ((g * _BM + rowT) < _S).astype(jnp.bfloat16)
    probs_bf = (p / l).astype(jnp.bfloat16)
    bs = jax.lax.dot_general(
        validT, probs_bf, (((1,), (0,)), ((), ())),
        preferred_element_type=jnp.float32)

    kidx = jax.lax.broadcasted_iota(jnp.int32, (1, _S), 1)
    bs_i = jax.lax.bitcast_convert_type(bs, jnp.int32)
    maxb = jnp.max(bs_i)

    def vbody(_, carry):
        lo, hi = carry
        mid = lo + ((hi - lo + 1) >> 1)
        cnt = jnp.sum((bs_i >= mid).astype(jnp.int32))
        ok = cnt >= _TOPK
        return jnp.where(ok, mid, lo), jnp.where(ok, hi, mid)

    t, _ = jax.lax.fori_loop(0, 31, vbody, (jnp.int32(0), maxb + 1))
    c_gt = jnp.sum((bs_i >= t + 1).astype(jnp.int32))
    quota = _TOPK - c_gt
    eq = bs_i == t

    def ibody(_, carry):
        lo, hi = carry
        mid = (lo + hi) >> 1
        cnt = jnp.sum((eq & (kidx <= mid)).astype(jnp.int32))
        ok = cnt >= quota
        return jnp.where(ok, lo, mid), jnp.where(ok, mid, hi)

    _, ncut = jax.lax.fori_loop(0, 11, ibody, (jnp.int32(-1), jnp.int32(_S - 1)))
    topk_mask = (bs_i >= t + 1) | (eq & (kidx <= ncut))

    gstart = g * _BM
    static = (kidx >= gstart - _LOCAL_W) & (kidx < gstart + _BM + _LOCAL_W)
    rmask = rm_ref[0, 0] != 0
    mask = topk_mask | rmask | static

    w = jnp.where(mask, p, 0.0)
    pmax = jnp.max(w, axis=-1, keepdims=True)
    p2 = w * (1.0 / pmax)
    l2 = jnp.sum(p2, axis=-1, keepdims=True)
    o_sparse = jax.lax.dot_general(
        p2.astype(jnp.bfloat16), vb, (((1,), (0,)), ((), ())),
        preferred_element_type=jnp.float32) / l2
    oc_ref[0, 0] = o - o_sparse


def kernel(q, k, v):
    b, h, s, d = q.shape
    rkey = jax.random.fold_in(jax.random.key(1), 7)
    rmask = (jax.random.uniform(rkey, (b, h, _G, s)) < _RAND_P)
    rmask = rmask[0].reshape(h, _G, 1, s).astype(jnp.int32)

    qp = jnp.pad(q, ((0, 0), (0, 0), (0, _SP - s), (0, 0)))

    o, oc = pl.pallas_call(
        _fused_kernel,
        grid=(h, _G),
        in_specs=[
            pl.BlockSpec((1, 1, _BM, d), lambda hh, gg: (0, hh, gg, 0)),
            pl.BlockSpec((1, 1, s, d), lambda hh, gg: (0, hh, 0, 0)),
            pl.BlockSpec((1, 1, s, d), lambda hh, gg: (0, hh, 0, 0)),
            pl.BlockSpec((1, 1, 1, s), lambda hh, gg: (hh, gg, 0, 0)),
        ],
        out_specs=[
            pl.BlockSpec((1, 1, _BM, d), lambda hh, gg: (0, hh, gg, 0)),
            pl.BlockSpec((1, 1, _BM, d), lambda hh, gg: (0, hh, gg, 0)),
        ],
        out_shape=[
            jax.ShapeDtypeStruct((b, h, _SP, d), jnp.float32),
            jax.ShapeDtypeStruct((b, h, _SP, d), jnp.float32),
        ],
        compiler_params=pltpu.CompilerParams(
            dimension_semantics=("parallel", "parallel")),
    )(qp, k, v, rmask)

    return jnp.stack([o[:, :, :s], oc[:, :, :s]], axis=0)


# speedup vs baseline: 1.0193x; 1.0193x over previous
import math

import jax
import jax.numpy as jnp
from jax.experimental import pallas as pl
from jax.experimental.pallas import tpu as pltpu

_B, _H, _S, _D = 1, 16, 2048, 128
_BM = 192
_TOPK = 512
_RAND_P = 0.01
_LOCAL_W = 128
_G = -(-_S // _BM)
_SP = _G * _BM
_CH = 256


def _fused_kernel(q_ref, k_ref, v_ref, rm_ref, o_ref, oc_ref):
    g = pl.program_id(1)
    q = q_ref[0, 0]
    k = k_ref[0, 0]
    v = v_ref[0, 0]
    scale = 1.0 / math.sqrt(_D)

    logits = jax.lax.dot_general(
        q.astype(jnp.bfloat16), k.astype(jnp.bfloat16),
        (((1,), (1,)), ((), ())),
        preferred_element_type=jnp.float32) * scale
    m = jnp.max(logits, axis=-1, keepdims=True)
    p = jnp.exp(logits - m)
    l = jnp.sum(p, axis=-1, keepdims=True)
    vb = v.astype(jnp.bfloat16)
    o = jax.lax.dot_general(
        p.astype(jnp.bfloat16), vb, (((1,), (0,)), ((), ())),
        preferred_element_type=jnp.float32) / l
    o_ref[0, 0] = o

    rowT = jax.lax.broadcasted_iota(jnp.int32, (1, _BM), 1)
    validT = ((g * _BM + rowT) < _S).astype(jnp.bfloat16)
    probs_bf = (p / l).astype(jnp.bfloat16)
    bs = jax.lax.dot_general(
        validT, probs_bf, (((1,), (0,)), ((), ())),
        preferred_element_type=jnp.float32)

    kidx = jax.lax.broadcasted_iota(jnp.int32, (1, _S), 1)
    bs_i = jax.lax.bitcast_convert_type(bs, jnp.int32)
    lo = jnp.zeros((1, 1), jnp.int32)
    hi = jnp.max(bs_i, keepdims=True) + 1
    for _ in range(31):
        mid = lo + ((hi - lo + 1) >> 1)
        cnt = jnp.sum((bs_i >= mid).astype(jnp.int32), keepdims=True)
        ok = cnt >= _TOPK
        lo = jnp.where(ok, mid, lo)
        hi = jnp.where(ok, hi, mid)
    t = lo
    c_gt = jnp.sum((bs_i >= t + 1).astype(jnp.int32), keepdims=True)
    quota = _TOPK - c_gt
    eq = bs_i == t
    ilo = jnp.full((1, 1), -1, jnp.int32)
    ihi = jnp.full((1, 1), _S - 1, jnp.int32)
    for _ in range(11):
        mid = (ilo + ihi) >> 1
        cnt = jnp.sum((eq & (kidx <= mid)).astype(jnp.int32), keepdims=True)
        ok = cnt >= quota
        ilo = jnp.where(ok, ilo, mid)
        ihi = jnp.where(ok, mid, ihi)
    topk_mask = (bs_i >= t + 1) | (eq & (kidx <= ihi))

    gstart = g * _BM
    static = (kidx >= gstart - _LOCAL_W) & (kidx < gstart + _BM + _LOCAL_W)
    rmask = rm_ref[0, 0] != 0
    mask = topk_mask | rmask | static

    w = jnp.where(mask, p, 0.0)
    pmax = jnp.max(w, axis=-1, keepdims=True)
    p2 = w * (1.0 / pmax)
    l2 = jnp.sum(p2, axis=-1, keepdims=True)
    o_sparse = jax.lax.dot_general(
        p2.astype(jnp.bfloat16), vb, (((1,), (0,)), ((), ())),
        preferred_element_type=jnp.float32) / l2
    oc_ref[0, 0] = o - o_sparse


def kernel(q, k, v):
    b, h, s, d = q.shape
    rkey = jax.random.fold_in(jax.random.key(1), 7)
    rmask = (jax.random.uniform(rkey, (b, h, _G, s)) < _RAND_P)
    rmask = rmask[0].reshape(h, _G, 1, s).astype(jnp.int32)

    qp = jnp.pad(q, ((0, 0), (0, 0), (0, _SP - s), (0, 0)))

    o, oc = pl.pallas_call(
        _fused_kernel,
        grid=(h, _G),
        in_specs=[
            pl.BlockSpec((1, 1, _BM, d), lambda hh, gg: (0, hh, gg, 0)),
            pl.BlockSpec((1, 1, s, d), lambda hh, gg: (0, hh, 0, 0)),
            pl.BlockSpec((1, 1, s, d), lambda hh, gg: (0, hh, 0, 0)),
            pl.BlockSpec((1, 1, 1, s), lambda hh, gg: (hh, gg, 0, 0)),
        ],
        out_specs=[
            pl.BlockSpec((1, 1, _BM, d), lambda hh, gg: (0, hh, gg, 0)),
            pl.BlockSpec((1, 1, _BM, d), lambda hh, gg: (0, hh, gg, 0)),
        ],
        out_shape=[
            jax.ShapeDtypeStruct((b, h, _SP, d), jnp.float32),
            jax.ShapeDtypeStruct((b, h, _SP, d), jnp.float32),
        ],
        compiler_params=pltpu.CompilerParams(
            dimension_semantics=("parallel", "parallel")),
    )(qp, k, v, rmask)

    return jnp.stack([o[:, :, :s], oc[:, :, :s]], axis=0)


# speedup vs baseline: 2.0415x; 2.0028x over previous
import math

import jax
import jax.numpy as jnp
from jax.experimental import pallas as pl
from jax.experimental.pallas import tpu as pltpu

_B, _H, _S, _D = 1, 16, 2048, 128
_BM = 192
_TOPK = 512
_RAND_P = 0.01
_LOCAL_W = 128
_G = -(-_S // _BM)
_SP = _G * _BM
_CH = 256


def _fused_kernel(q_ref, k_ref, v_ref, rm_ref, o_ref, oc_ref):
    g = pl.program_id(1)
    q = q_ref[0, 0]
    k = k_ref[0, 0]
    v = v_ref[0, 0]
    scale = 1.0 / math.sqrt(_D)

    logits = jax.lax.dot_general(
        q.astype(jnp.bfloat16), k.astype(jnp.bfloat16),
        (((1,), (1,)), ((), ())),
        preferred_element_type=jnp.float32) * scale
    m = jnp.max(logits, axis=-1, keepdims=True)
    p = jnp.exp(logits - m)
    l = jnp.sum(p, axis=-1, keepdims=True)
    vb = v.astype(jnp.bfloat16)
    o = jax.lax.dot_general(
        p.astype(jnp.bfloat16), vb, (((1,), (0,)), ((), ())),
        preferred_element_type=jnp.float32) / l
    o_ref[0, 0] = o

    rowT = jax.lax.broadcasted_iota(jnp.int32, (1, _BM), 1)
    validT = ((g * _BM + rowT) < _S).astype(jnp.bfloat16)
    probs_bf = (p / l).astype(jnp.bfloat16)
    bs = jax.lax.dot_general(
        validT, probs_bf, (((1,), (0,)), ((), ())),
        preferred_element_type=jnp.float32)

    kidx = jax.lax.broadcasted_iota(jnp.int32, (1, _S), 1)
    bs_i = jax.lax.bitcast_convert_type(bs, jnp.int32)
    k16 = jax.lax.broadcasted_iota(jnp.int32, (16, 1), 0) + 1
    lo = jnp.zeros((1, 1), jnp.int32)
    hi = jnp.max(bs_i, keepdims=True) + 1
    for _ in range(9):
        step = (hi - lo + 15) >> 4
        thr = lo + k16 * step
        cnt = jnp.sum((bs_i >= thr).astype(jnp.int32), axis=1, keepdims=True)
        s = jnp.sum((cnt >= _TOPK).astype(jnp.int32), keepdims=True)
        lo = lo + s * step
        hi = lo + step
    t = lo
    c_gt = jnp.sum((bs_i >= t + 1).astype(jnp.int32), keepdims=True)
    quota = _TOPK - c_gt
    eq = bs_i == t
    ilo = jnp.full((1, 1), -1, jnp.int32)
    ihi = jnp.full((1, 1), _S - 1, jnp.int32)
    for _ in range(3):
        step = (ihi - ilo + 15) >> 4
        thr = ilo + k16 * step
        cnt = jnp.sum((eq & (kidx <= thr)).astype(jnp.int32), axis=1, keepdims=True)
        s = jnp.sum((cnt < quota).astype(jnp.int32), keepdims=True)
        ilo = ilo + s * step
        ihi = ilo + step
    topk_mask = (bs_i >= t + 1) | (eq & (kidx <= ihi))

    gstart = g * _BM
    static = (kidx >= gstart - _LOCAL_W) & (kidx < gstart + _BM + _LOCAL_W)
    rmask = rm_ref[0, 0] != 0
    mask = topk_mask | rmask | static

    w = jnp.where(mask, p, 0.0)
    pmax = jnp.max(w, axis=-1, keepdims=True)
    p2 = w * (1.0 / pmax)
    l2 = jnp.sum(p2, axis=-1, keepdims=True)
    o_sparse = jax.lax.dot_general(
        p2.astype(jnp.bfloat16), vb, (((1,), (0,)), ((), ())),
        preferred_element_type=jnp.float32) / l2
    oc_ref[0, 0] = o - o_sparse


def kernel(q, k, v):
    b, h, s, d = q.shape
    rkey = jax.random.fold_in(jax.random.key(1), 7)
    rmask = (jax.random.uniform(rkey, (b, h, _G, s)) < _RAND_P)
    rmask = rmask[0].reshape(h, _G, 1, s).astype(jnp.int32)

    qp = jnp.pad(q, ((0, 0), (0, 0), (0, _SP - s), (0, 0)))

    o, oc = pl.pallas_call(
        _fused_kernel,
        grid=(h, _G),
        in_specs=[
            pl.BlockSpec((1, 1, _BM, d), lambda hh, gg: (0, hh, gg, 0)),
            pl.BlockSpec((1, 1, s, d), lambda hh, gg: (0, hh, 0, 0)),
            pl.BlockSpec((1, 1, s, d), lambda hh, gg: (0, hh, 0, 0)),
            pl.BlockSpec((1, 1, 1, s), lambda hh, gg: (hh, gg, 0, 0)),
        ],
        out_specs=[
            pl.BlockSpec((1, 1, _BM, d), lambda hh, gg: (0, hh, gg, 0)),
            pl.BlockSpec((1, 1, _BM, d), lambda hh, gg: (0, hh, gg, 0)),
        ],
        out_shape=[
            jax.ShapeDtypeStruct((b, h, _SP, d), jnp.float32),
            jax.ShapeDtypeStruct((b, h, _SP, d), jnp.float32),
        ],
        compiler_params=pltpu.CompilerParams(
            dimension_semantics=("parallel", "parallel")),
    )(qp, k, v, rmask)

    return jnp.stack([o[:, :, :s], oc[:, :, :s]], axis=0)
